# Initial kernel scaffold; baseline (speedup 1.0000x reference)
import jax
import jax.numpy as jnp
from jax.experimental import pallas as pl


def kernel(x, edge_index, Wl1, bl1, Wr1, Wl2, bl2, Wr2, Wfc, bfc):
    raise NotImplementedError("write your pallas kernel here")


# speedup vs baseline: 8.4676x; 8.4676x over previous
import functools

import jax
import jax.numpy as jnp
from jax import lax
from jax.experimental import pallas as pl
from jax.experimental.pallas import tpu as pltpu
import jax.experimental.pallas.tpu_sc as plsc

N = 10000
E = 320000
NC, NS, L = 2, 16, 16
CH = 128
EPT = 10112
EPAD = EPT * NC * NS
N_SH = NS * 640
BR = 1000


def _edge_agg_body(with_cnt, D, y_hbm, src_hbm, dst_hbm, *refs):
  if with_cnt:
    (out_hbm, cnt_hbm, src_v, dst_v, rows_v, z_v, ones_v,
     shared, shared_cnt, ob_v, oc_v, sem) = refs
  else:
    (out_hbm, src_v, dst_v, rows_v, z_v, shared, ob_v, sem) = refs
  c = lax.axis_index("c")
  s = lax.axis_index("s")

  def zrow(i, _):
    for j in range(D // L):
      z_v[i, pl.ds(j * L, L)] = jnp.zeros((L,), jnp.float32)
    if with_cnt:
      ones_v[i, pl.ds(0, L)] = jnp.zeros((L,), jnp.float32)
    return 0
  lax.fori_loop(0, CH, zrow, 0, unroll=4)

  r0 = s * 640
  def icopy(k, _):
    pltpu.sync_copy(z_v, shared.at[pl.ds(r0 + k * CH, CH)])
    if with_cnt:
      pltpu.sync_copy(ones_v, shared_cnt.at[pl.ds(r0 + k * CH, CH)])
    return 0
  lax.fori_loop(0, 640 // CH, icopy, 0)
  if with_cnt:
    def onesrow(i, _):
      ones_v[i, pl.ds(0, L)] = jnp.ones((L,), jnp.float32)
      return 0
    lax.fori_loop(0, CH, onesrow, 0, unroll=4)
  plsc.subcore_barrier()

  ebase = (c * NS + s) * EPT
  def echunk(j, _):
    off = ebase + j * CH
    pltpu.sync_copy(src_hbm.at[pl.ds(off, CH)], src_v)
    pltpu.sync_copy(dst_hbm.at[pl.ds(off, CH)], dst_v)
    pltpu.async_copy(y_hbm.at[src_v], rows_v, sem).wait()
    pltpu.sync_copy(rows_v, shared.at[dst_v], add=True)
    if with_cnt:
      pltpu.sync_copy(ones_v, shared_cnt.at[dst_v], add=True)
    return 0
  lax.fori_loop(0, EPT // CH, echunk, 0)
  plsc.subcore_barrier()

  pltpu.sync_copy(shared.at[pl.ds(r0, 640)], ob_v)
  pltpu.sync_copy(ob_v, out_hbm.at[c, pl.ds(r0, 640)])
  if with_cnt:
    pltpu.sync_copy(shared_cnt.at[pl.ds(r0, 640)], oc_v)
    pltpu.sync_copy(oc_v, cnt_hbm.at[c, pl.ds(r0, 640)])


def _make_edge_agg(D, with_cnt):
  mesh = plsc.VectorSubcoreMesh(core_axis_name="c", subcore_axis_name="s",
                                num_cores=NC, num_subcores=NS)
  out_type = [jax.ShapeDtypeStruct((NC, N_SH, D), jnp.float32)]
  scratch = [
      pltpu.VMEM((CH,), jnp.int32),
      pltpu.VMEM((CH,), jnp.int32),
      pltpu.VMEM((CH, D), jnp.float32),
      pltpu.VMEM((CH, D), jnp.float32),
  ]
  if with_cnt:
    out_type.append(jax.ShapeDtypeStruct((NC, N_SH, L), jnp.float32))
    scratch.append(pltpu.VMEM((CH, L), jnp.float32))
  scratch.append(pltpu.VMEM_SHARED((N_SH, D), jnp.float32))
  if with_cnt:
    scratch.append(pltpu.VMEM_SHARED((N_SH, L), jnp.float32))
  scratch.append(pltpu.VMEM((640, D), jnp.float32))
  if with_cnt:
    scratch.append(pltpu.VMEM((640, L), jnp.float32))
  scratch.append(pltpu.SemaphoreType.DMA)
  return pl.kernel(
      functools.partial(_edge_agg_body, with_cnt, D),
      out_type=out_type, mesh=mesh, scratch_types=scratch,
      compiler_params=pltpu.CompilerParams(use_tc_tiling_on_sc=False),
      name=f"edge_agg_d{D}")


def _tc_a_body(x_ref, wl_ref, wr_ref, bl_ref, y_ref, r_ref):
  xb = x_ref[...]
  y_ref[...] = jnp.dot(xb, wl_ref[...], preferred_element_type=jnp.float32)
  r_ref[...] = (jnp.dot(xb, wr_ref[...], preferred_element_type=jnp.float32)
                + bl_ref[...])


def _tc_mid_body(p_ref, c_ref, r_ref, wl_ref, wr_ref, bl_ref,
                 y_ref, r2_ref):
  agg = p_ref[0] + p_ref[1]
  cnt = (c_ref[0] + c_ref[1])[:, 0:1]
  mean = agg / jnp.maximum(cnt, 1.0)
  h = jax.nn.relu(mean + r_ref[...])
  y_ref[...] = jnp.dot(h, wl_ref[...], preferred_element_type=jnp.float32)
  r2_ref[...] = (jnp.dot(h, wr_ref[...], preferred_element_type=jnp.float32)
                 + bl_ref[...])


def _tc_out_body(p_ref, c_ref, r_ref, wfc_ref, bfc_ref, o_ref):
  agg = p_ref[0] + p_ref[1]
  cnt = (c_ref[0] + c_ref[1])[:, 0:1]
  mean = agg / jnp.maximum(cnt, 1.0)
  h = jax.nn.relu(mean + r_ref[...])
  o_ref[...] = (jnp.dot(h, wfc_ref[...], preferred_element_type=jnp.float32)
                + bfc_ref[...])


def _row_spec(d):
  return pl.BlockSpec((BR, d), lambda i: (i, 0))


def _part_spec(d):
  return pl.BlockSpec((NC, BR, d), lambda i: (0, i, 0))


def _full_spec(a, b):
  return pl.BlockSpec((a, b), lambda i: (0, 0))


def kernel(x, edge_index, Wl1, bl1, Wr1, Wl2, bl2, Wr2, Wfc, bfc):
  ei = edge_index.astype(jnp.int32)
  pad = EPAD - E
  src = jnp.concatenate([ei[0], jnp.zeros((pad,), jnp.int32)])
  dst = jnp.concatenate([ei[1], jnp.full((pad,), N, jnp.int32)])

  grid = N // BR

  y1, r1 = pl.pallas_call(
      _tc_a_body,
      grid=(grid,),
      in_specs=[_row_spec(128), _full_spec(128, 32), _full_spec(128, 32),
                _full_spec(1, 32)],
      out_specs=[_row_spec(32), _row_spec(32)],
      out_shape=[jax.ShapeDtypeStruct((N, 32), jnp.float32)] * 2,
  )(x, Wl1.T, Wr1.T, bl1.reshape(1, 32))

  p1, cnt = _make_edge_agg(32, True)(y1, src, dst)

  y2, r2 = pl.pallas_call(
      _tc_mid_body,
      grid=(grid,),
      in_specs=[_part_spec(32), _part_spec(L), _row_spec(32),
                _full_spec(32, 16), _full_spec(32, 16), _full_spec(1, 16)],
      out_specs=[_row_spec(16), _row_spec(16)],
      out_shape=[jax.ShapeDtypeStruct((N, 16), jnp.float32)] * 2,
  )(p1, cnt, r1, Wl2.T, Wr2.T, bl2.reshape(1, 16))

  (p2,) = _make_edge_agg(16, False)(y2, src, dst)

  out = pl.pallas_call(
      _tc_out_body,
      grid=(grid,),
      in_specs=[_part_spec(16), _part_spec(L), _row_spec(16),
                _full_spec(16, 2), _full_spec(1, 2)],
      out_specs=_row_spec(2),
      out_shape=jax.ShapeDtypeStruct((N, 2), jnp.float32),
  )(p2, cnt, r2, Wfc.T, bfc.reshape(1, 2))
  return out


# speedup vs baseline: 13.9535x; 1.6479x over previous
import functools

import jax
import jax.numpy as jnp
from jax import lax
from jax.experimental import pallas as pl
from jax.experimental.pallas import tpu as pltpu
import jax.experimental.pallas.tpu_sc as plsc

N = 10000
E = 320000
NC, NS, L = 2, 16, 16
CH = 128
CPB = 8
GB = 10
EPT = CH * CPB * GB
EPAD = EPT * NC * NS
N_SH = NS * 640
BR = 1000


def _edge_agg_body(with_cnt, D, y_hbm, src_hbm, dst_hbm, *refs):
  if with_cnt:
    (out_hbm, cnt_hbm, srcw_v, dstw_v, rows_v, z_v, z16_v, ones_v,
     shared, shared_cnt) = refs[:10]
  else:
    (out_hbm, srcw_v, dstw_v, rows_v, z_v, shared) = refs[:6]
  sems = refs[-CPB:]
  c = lax.axis_index("c")
  s = lax.axis_index("s")
  w = c * NS + s

  pltpu.sync_copy(src_hbm.at[w], srcw_v)
  pltpu.sync_copy(dst_hbm.at[w], dstw_v)
  for b in range(CPB):
    pltpu.async_copy(y_hbm.at[srcw_v.at[b]], rows_v.at[b], sems[b])

  def zrow(i, _):
    for j in range(D // L):
      z_v[i, pl.ds(j * L, L)] = jnp.zeros((L,), jnp.float32)
    if with_cnt:
      z16_v[i, pl.ds(0, L)] = jnp.zeros((L,), jnp.float32)
      ones_v[i, pl.ds(0, L)] = jnp.ones((L,), jnp.float32)
    return 0
  lax.fori_loop(0, CH, zrow, 0, unroll=4)

  r0 = s * 640
  def icopy(k, _):
    pltpu.sync_copy(z_v, shared.at[pl.ds(r0 + k * CH, CH)])
    if with_cnt:
      pltpu.sync_copy(z16_v, shared_cnt.at[pl.ds(r0 + k * CH, CH)])
    return 0
  lax.fori_loop(0, 640 // CH, icopy, 0)
  plsc.subcore_barrier()

  def gwait(b):
    pltpu.make_async_copy(y_hbm.at[srcw_v.at[0]], rows_v.at[b],
                          sems[b]).wait()

  def swait(b):
    pltpu.make_async_copy(rows_v.at[b], shared.at[dstw_v.at[0]],
                          sems[b]).wait()
    if with_cnt:
      pltpu.make_async_copy(ones_v, shared_cnt.at[dstw_v.at[0]],
                            sems[b]).wait()

  def fire_scatter(b, j):
    pltpu.async_copy(rows_v.at[b], shared.at[dstw_v.at[j]], sems[b],
                     add=True)
    if with_cnt:
      pltpu.async_copy(ones_v, shared_cnt.at[dstw_v.at[j]], sems[b],
                       add=True)

  def pbody(g, _):
    for b in range(CPB):
      gwait(b)
      fire_scatter(b, g * CPB + b)
    for b in range(CPB):
      swait(b)
      pltpu.async_copy(y_hbm.at[srcw_v.at[(g + 1) * CPB + b]],
                       rows_v.at[b], sems[b])
    return 0
  lax.fori_loop(0, GB - 1, pbody, 0)
  for b in range(CPB):
    gwait(b)
    fire_scatter(b, (GB - 1) * CPB + b)
  for b in range(CPB):
    swait(b)
  plsc.subcore_barrier()

  for k in range(640 // CH):
    pltpu.sync_copy(shared.at[pl.ds(r0 + k * CH, CH)], rows_v.at[0])
    pltpu.sync_copy(rows_v.at[0], out_hbm.at[c, pl.ds(r0 + k * CH, CH)])
  if with_cnt:
    for k in range(640 // CH):
      pltpu.sync_copy(shared_cnt.at[pl.ds(r0 + k * CH, CH)], ones_v)
      pltpu.sync_copy(ones_v, cnt_hbm.at[c, pl.ds(r0 + k * CH, CH)])


def _make_edge_agg(D, with_cnt):
  mesh = plsc.VectorSubcoreMesh(core_axis_name="c", subcore_axis_name="s",
                                num_cores=NC, num_subcores=NS)
  out_type = [jax.ShapeDtypeStruct((NC, N_SH, D), jnp.float32)]
  scratch = [
      pltpu.VMEM((GB * CPB, CH), jnp.int32),
      pltpu.VMEM((GB * CPB, CH), jnp.int32),
      pltpu.VMEM((CPB, CH, D), jnp.float32),
      pltpu.VMEM((CH, D), jnp.float32),
  ]
  if with_cnt:
    out_type.append(jax.ShapeDtypeStruct((NC, N_SH, L), jnp.float32))
    scratch.append(pltpu.VMEM((CH, L), jnp.float32))
    scratch.append(pltpu.VMEM((CH, L), jnp.float32))
  scratch.append(pltpu.VMEM_SHARED((N_SH, D), jnp.float32))
  if with_cnt:
    scratch.append(pltpu.VMEM_SHARED((N_SH, L), jnp.float32))
  scratch.extend([pltpu.SemaphoreType.DMA] * CPB)
  return pl.kernel(
      functools.partial(_edge_agg_body, with_cnt, D),
      out_type=out_type, mesh=mesh, scratch_types=scratch,
      compiler_params=pltpu.CompilerParams(use_tc_tiling_on_sc=False),
      name=f"edge_agg_d{D}")


def _tc_a_body(x_ref, wl_ref, wr_ref, bl_ref, y_ref, r_ref):
  xb = x_ref[...]
  y_ref[...] = jnp.dot(xb, wl_ref[...], preferred_element_type=jnp.float32)
  r_ref[...] = (jnp.dot(xb, wr_ref[...], preferred_element_type=jnp.float32)
                + bl_ref[...])


def _tc_mid_body(p_ref, c_ref, r_ref, wl_ref, wr_ref, bl_ref,
                 y_ref, r2_ref):
  agg = p_ref[0] + p_ref[1]
  cnt = (c_ref[0] + c_ref[1])[:, 0:1]
  mean = agg / jnp.maximum(cnt, 1.0)
  h = jax.nn.relu(mean + r_ref[...])
  y_ref[...] = jnp.dot(h, wl_ref[...], preferred_element_type=jnp.float32)
  r2_ref[...] = (jnp.dot(h, wr_ref[...], preferred_element_type=jnp.float32)
                 + bl_ref[...])


def _tc_out_body(p_ref, c_ref, r_ref, wfc_ref, bfc_ref, o_ref):
  agg = p_ref[0] + p_ref[1]
  cnt = (c_ref[0] + c_ref[1])[:, 0:1]
  mean = agg / jnp.maximum(cnt, 1.0)
  h = jax.nn.relu(mean + r_ref[...])
  o_ref[...] = (jnp.dot(h, wfc_ref[...], preferred_element_type=jnp.float32)
                + bfc_ref[...])


def _row_spec(d):
  return pl.BlockSpec((BR, d), lambda i: (i, 0))


def _part_spec(d):
  return pl.BlockSpec((NC, BR, d), lambda i: (0, i, 0))


def _full_spec(a, b):
  return pl.BlockSpec((a, b), lambda i: (0, 0))


def kernel(x, edge_index, Wl1, bl1, Wr1, Wl2, bl2, Wr2, Wfc, bfc):
  ei = edge_index.astype(jnp.int32)
  pad = EPAD - E
  src = jnp.concatenate([ei[0], jnp.zeros((pad,), jnp.int32)])
  src = src.reshape(NC * NS, GB * CPB, CH)
  dst = jnp.concatenate([ei[1], jnp.full((pad,), N, jnp.int32)])
  dst = dst.reshape(NC * NS, GB * CPB, CH)

  grid = N // BR

  y1, r1 = pl.pallas_call(
      _tc_a_body,
      grid=(grid,),
      in_specs=[_row_spec(128), _full_spec(128, 32), _full_spec(128, 32),
                _full_spec(1, 32)],
      out_specs=[_row_spec(32), _row_spec(32)],
      out_shape=[jax.ShapeDtypeStruct((N, 32), jnp.float32)] * 2,
  )(x, Wl1.T, Wr1.T, bl1.reshape(1, 32))

  p1, cnt = _make_edge_agg(32, True)(y1, src, dst)

  y2, r2 = pl.pallas_call(
      _tc_mid_body,
      grid=(grid,),
      in_specs=[_part_spec(32), _part_spec(L), _row_spec(32),
                _full_spec(32, 16), _full_spec(32, 16), _full_spec(1, 16)],
      out_specs=[_row_spec(16), _row_spec(16)],
      out_shape=[jax.ShapeDtypeStruct((N, 16), jnp.float32)] * 2,
  )(p1, cnt, r1, Wl2.T, Wr2.T, bl2.reshape(1, 16))

  (p2,) = _make_edge_agg(16, False)(y2, src, dst)

  out = pl.pallas_call(
      _tc_out_body,
      grid=(grid,),
      in_specs=[_part_spec(16), _part_spec(L), _row_spec(16),
                _full_spec(16, 2), _full_spec(1, 2)],
      out_specs=_row_spec(2),
      out_shape=jax.ShapeDtypeStruct((N, 2), jnp.float32),
  )(p2, cnt, r2, Wfc.T, bfc.reshape(1, 2))
  return out


# speedup vs baseline: 13.9606x; 1.0005x over previous
import functools

import jax
import jax.numpy as jnp
from jax import lax
from jax.experimental import pallas as pl
from jax.experimental.pallas import tpu as pltpu
import jax.experimental.pallas.tpu_sc as plsc

N = 10000
E = 320000
NC, NS, L = 2, 16, 16
CH = 128
CPB = 8
GB = 10
EPT = CH * CPB * GB
EPAD = EPT * NC * NS
N_SH = NS * 640
BR = 1000


def _edge_agg_body(with_cnt, D, y_hbm, src_hbm, dst_hbm, *refs):
  if with_cnt:
    (out_hbm, cnt_hbm, srcw_v, dstw_v, rows_v, z_v, z16_v, ones_v,
     shared, shared_cnt) = refs[:10]
  else:
    (out_hbm, srcw_v, dstw_v, rows_v, z_v, shared) = refs[:6]
  sems = refs[-CPB:]
  c = lax.axis_index("c")
  s = lax.axis_index("s")
  w = c * NS + s

  pltpu.sync_copy(src_hbm.at[w], srcw_v)
  pltpu.sync_copy(dst_hbm.at[w], dstw_v)
  for b in range(CPB):
    pltpu.async_copy(y_hbm.at[srcw_v.at[b]], rows_v.at[b], sems[b])

  def zrow(i, _):
    for j in range(D // L):
      z_v[i, pl.ds(j * L, L)] = jnp.zeros((L,), jnp.float32)
    if with_cnt:
      z16_v[i, pl.ds(0, L)] = jnp.zeros((L,), jnp.float32)
      ones_v[i, pl.ds(0, L)] = jnp.ones((L,), jnp.float32)
    return 0
  lax.fori_loop(0, CH, zrow, 0, unroll=4)

  r0 = s * 640
  def icopy(k, _):
    pltpu.sync_copy(z_v, shared.at[pl.ds(r0 + k * CH, CH)])
    if with_cnt:
      pltpu.sync_copy(z16_v, shared_cnt.at[pl.ds(r0 + k * CH, CH)])
    return 0
  lax.fori_loop(0, 640 // CH, icopy, 0)
  plsc.subcore_barrier()

  def gwait(b):
    pltpu.make_async_copy(y_hbm.at[srcw_v.at[0]], rows_v.at[b],
                          sems[b]).wait()

  def swait(b):
    pltpu.make_async_copy(rows_v.at[b], shared.at[dstw_v.at[0]],
                          sems[b]).wait()
    if with_cnt:
      pltpu.make_async_copy(ones_v, shared_cnt.at[dstw_v.at[0]],
                            sems[b]).wait()

  def fire_scatter(b, j):
    pltpu.async_copy(rows_v.at[b], shared.at[dstw_v.at[j]], sems[b],
                     add=True)
    if with_cnt:
      pltpu.async_copy(ones_v, shared_cnt.at[dstw_v.at[j]], sems[b],
                       add=True)

  def pbody(g, _):
    for b in range(CPB):
      gwait(b)
      fire_scatter(b, g * CPB + b)
    for b in range(CPB):
      swait(b)
      pltpu.async_copy(y_hbm.at[srcw_v.at[(g + 1) * CPB + b]],
                       rows_v.at[b], sems[b])
    return 0
  lax.fori_loop(0, GB - 1, pbody, 0)
  for b in range(CPB):
    gwait(b)
    fire_scatter(b, (GB - 1) * CPB + b)
  for b in range(CPB):
    swait(b)
  plsc.subcore_barrier()

  for k in range(640 // CH):
    pltpu.sync_copy(shared.at[pl.ds(r0 + k * CH, CH)], rows_v.at[0])
    pltpu.sync_copy(rows_v.at[0], out_hbm.at[c, pl.ds(r0 + k * CH, CH)])
  if with_cnt:
    for k in range(640 // CH):
      pltpu.sync_copy(shared_cnt.at[pl.ds(r0 + k * CH, CH)], ones_v)
      pltpu.sync_copy(ones_v, cnt_hbm.at[c, pl.ds(r0 + k * CH, CH)])


def _make_edge_agg(D, with_cnt):
  mesh = plsc.VectorSubcoreMesh(core_axis_name="c", subcore_axis_name="s",
                                num_cores=NC, num_subcores=NS)
  out_type = [jax.ShapeDtypeStruct((NC, N_SH, D), jnp.float32)]
  scratch = [
      pltpu.VMEM((GB * CPB, CH), jnp.int32),
      pltpu.VMEM((GB * CPB, CH), jnp.int32),
      pltpu.VMEM((CPB, CH, D), jnp.float32),
      pltpu.VMEM((CH, D), jnp.float32),
  ]
  if with_cnt:
    out_type.append(jax.ShapeDtypeStruct((NC, N_SH, L), jnp.float32))
    scratch.append(pltpu.VMEM((CH, L), jnp.float32))
    scratch.append(pltpu.VMEM((CH, L), jnp.float32))
  scratch.append(pltpu.VMEM_SHARED((N_SH, D), jnp.float32))
  if with_cnt:
    scratch.append(pltpu.VMEM_SHARED((N_SH, L), jnp.float32))
  scratch.extend([pltpu.SemaphoreType.DMA] * CPB)
  return pl.kernel(
      functools.partial(_edge_agg_body, with_cnt, D),
      out_type=out_type, mesh=mesh, scratch_types=scratch,
      compiler_params=pltpu.CompilerParams(use_tc_tiling_on_sc=False),
      name=f"edge_agg_d{D}")


def _tc_a_body(x_ref, wl_ref, wr_ref, bl_ref, y_ref, r_ref):
  xb = x_ref[...]
  y_ref[...] = jnp.dot(xb, wl_ref[...], preferred_element_type=jnp.float32)
  r_ref[...] = (jnp.dot(xb, wr_ref[...], preferred_element_type=jnp.float32)
                + bl_ref[...])


def _tc_mid_body(p_ref, c_ref, r_ref, wl_ref, wr_ref, bl_ref,
                 y_ref, r2_ref):
  agg = p_ref[0] + p_ref[1]
  cnt = (c_ref[0] + c_ref[1])[:, 0:1]
  mean = agg / jnp.maximum(cnt, 1.0)
  h = jax.nn.relu(mean + r_ref[...])
  y_ref[...] = jnp.dot(h, wl_ref[...], preferred_element_type=jnp.float32)
  r2_ref[...] = (jnp.dot(h, wr_ref[...], preferred_element_type=jnp.float32)
                 + bl_ref[...])


def _tc_out_body(p_ref, c_ref, r_ref, wfc_ref, bfc_ref, o_ref):
  agg = p_ref[0] + p_ref[1]
  cnt = (c_ref[0] + c_ref[1])[:, 0:1]
  mean = agg / jnp.maximum(cnt, 1.0)
  h = jax.nn.relu(mean + r_ref[...])
  o_ref[...] = (jnp.dot(h, wfc_ref[...], preferred_element_type=jnp.float32)
                + bfc_ref[...])


def _row_spec(d):
  return pl.BlockSpec((BR, d), lambda i: (i, 0))


def _part_spec(d):
  return pl.BlockSpec((NC, BR, d), lambda i: (0, i, 0))


def _full_spec(a, b):
  return pl.BlockSpec((a, b), lambda i: (0, 0))


def kernel(x, edge_index, Wl1, bl1, Wr1, Wl2, bl2, Wr2, Wfc, bfc):
  ei = edge_index.astype(jnp.int32)
  pad = EPAD - E
  src = jnp.concatenate([ei[0], jnp.zeros((pad,), jnp.int32)])
  src = src.reshape(NC * NS, GB * CPB, CH)
  pad_dst = N + jnp.arange(pad, dtype=jnp.int32) % (N_SH - N)
  dst = jnp.concatenate([ei[1], pad_dst])
  dst = dst.reshape(NC * NS, GB * CPB, CH)

  grid = N // BR

  y1, r1 = pl.pallas_call(
      _tc_a_body,
      grid=(grid,),
      in_specs=[_row_spec(128), _full_spec(128, 32), _full_spec(128, 32),
                _full_spec(1, 32)],
      out_specs=[_row_spec(32), _row_spec(32)],
      out_shape=[jax.ShapeDtypeStruct((N, 32), jnp.float32)] * 2,
  )(x, Wl1.T, Wr1.T, bl1.reshape(1, 32))

  p1, cnt = _make_edge_agg(32, True)(y1, src, dst)

  y2, r2 = pl.pallas_call(
      _tc_mid_body,
      grid=(grid,),
      in_specs=[_part_spec(32), _part_spec(L), _row_spec(32),
                _full_spec(32, 16), _full_spec(32, 16), _full_spec(1, 16)],
      out_specs=[_row_spec(16), _row_spec(16)],
      out_shape=[jax.ShapeDtypeStruct((N, 16), jnp.float32)] * 2,
  )(p1, cnt, r1, Wl2.T, Wr2.T, bl2.reshape(1, 16))

  (p2,) = _make_edge_agg(16, False)(y2, src, dst)

  out = pl.pallas_call(
      _tc_out_body,
      grid=(grid,),
      in_specs=[_part_spec(16), _part_spec(L), _row_spec(16),
                _full_spec(16, 2), _full_spec(1, 2)],
      out_specs=_row_spec(2),
      out_shape=jax.ShapeDtypeStruct((N, 2), jnp.float32),
  )(p2, cnt, r2, Wfc.T, bfc.reshape(1, 2))
  return out


# speedup vs baseline: 14.1690x; 1.0149x over previous
import functools

import jax
import jax.numpy as jnp
from jax import lax
from jax.experimental import pallas as pl
from jax.experimental.pallas import tpu as pltpu
import jax.experimental.pallas.tpu_sc as plsc

N = 10000
E = 320000
NC, NS, L = 2, 16, 16
CH = 128
CPB = 8
NB0 = 15
NB1 = 5
NBMAX = max(NB0, NB1)
EPAD = CH * CPB * (NB0 + NB1) * NS
N_SH = NS * 640
BR = 1000


def _edge_agg_body(with_cnt, D, y_hbm, src_hbm, dst_hbm, *refs):
  if with_cnt:
    (out_hbm, cnt_hbm, srcw_v, dstw_v, rows_v, z_v, z16_v, ones_v,
     shared, shared_cnt) = refs[:10]
  else:
    (out_hbm, srcw_v, dstw_v, rows_v, z_v, shared) = refs[:6]
  sems = refs[-CPB:]
  c = lax.axis_index("c")
  s = lax.axis_index("s")
  w = c * NS + s

  pltpu.sync_copy(src_hbm.at[w], srcw_v)
  pltpu.sync_copy(dst_hbm.at[w], dstw_v)
  for b in range(CPB):
    pltpu.async_copy(y_hbm.at[srcw_v.at[b]], rows_v.at[b], sems[b])

  def zrow(i, _):
    for j in range(D // L):
      z_v[i, pl.ds(j * L, L)] = jnp.zeros((L,), jnp.float32)
    if with_cnt:
      z16_v[i, pl.ds(0, L)] = jnp.zeros((L,), jnp.float32)
      ones_v[i, pl.ds(0, L)] = jnp.ones((L,), jnp.float32)
    return 0
  lax.fori_loop(0, CH, zrow, 0, unroll=4)

  r0 = s * 640
  def icopy(k, _):
    pltpu.sync_copy(z_v, shared.at[pl.ds(r0 + k * CH, CH)])
    if with_cnt:
      pltpu.sync_copy(z16_v, shared_cnt.at[pl.ds(r0 + k * CH, CH)])
    return 0
  lax.fori_loop(0, 640 // CH, icopy, 0)
  plsc.subcore_barrier()

  def gwait(b):
    pltpu.make_async_copy(y_hbm.at[srcw_v.at[0]], rows_v.at[b],
                          sems[b]).wait()

  def swait(b):
    pltpu.make_async_copy(rows_v.at[b], shared.at[dstw_v.at[0]],
                          sems[b]).wait()
    if with_cnt:
      pltpu.make_async_copy(ones_v, shared_cnt.at[dstw_v.at[0]],
                            sems[b]).wait()

  def fire_scatter(b, j):
    pltpu.async_copy(rows_v.at[b], shared.at[dstw_v.at[j]], sems[b],
                     add=True)
    if with_cnt:
      pltpu.async_copy(ones_v, shared_cnt.at[dstw_v.at[j]], sems[b],
                       add=True)

  nb = jnp.where(c == 0, NB0, NB1)
  def pbody(g, _):
    for b in range(CPB):
      gwait(b)
      fire_scatter(b, g * CPB + b)
    for b in range(CPB):
      swait(b)
      pltpu.async_copy(y_hbm.at[srcw_v.at[(g + 1) * CPB + b]],
                       rows_v.at[b], sems[b])
    return 0
  lax.fori_loop(0, nb - 1, pbody, 0)
  for b in range(CPB):
    gwait(b)
    fire_scatter(b, (nb - 1) * CPB + b)
  for b in range(CPB):
    swait(b)
  plsc.subcore_barrier()

  for k in range(640 // CH):
    pltpu.sync_copy(shared.at[pl.ds(r0 + k * CH, CH)], rows_v.at[0])
    pltpu.sync_copy(rows_v.at[0], out_hbm.at[c, pl.ds(r0 + k * CH, CH)])
  if with_cnt:
    for k in range(640 // CH):
      pltpu.sync_copy(shared_cnt.at[pl.ds(r0 + k * CH, CH)], ones_v)
      pltpu.sync_copy(ones_v, cnt_hbm.at[c, pl.ds(r0 + k * CH, CH)])


def _make_edge_agg(D, with_cnt):
  mesh = plsc.VectorSubcoreMesh(core_axis_name="c", subcore_axis_name="s",
                                num_cores=NC, num_subcores=NS)
  out_type = [jax.ShapeDtypeStruct((NC, N_SH, D), jnp.float32)]
  scratch = [
      pltpu.VMEM((NBMAX * CPB, CH), jnp.int32),
      pltpu.VMEM((NBMAX * CPB, CH), jnp.int32),
      pltpu.VMEM((CPB, CH, D), jnp.float32),
      pltpu.VMEM((CH, D), jnp.float32),
  ]
  if with_cnt:
    out_type.append(jax.ShapeDtypeStruct((NC, N_SH, L), jnp.float32))
    scratch.append(pltpu.VMEM((CH, L), jnp.float32))
    scratch.append(pltpu.VMEM((CH, L), jnp.float32))
  scratch.append(pltpu.VMEM_SHARED((N_SH, D), jnp.float32))
  if with_cnt:
    scratch.append(pltpu.VMEM_SHARED((N_SH, L), jnp.float32))
  scratch.extend([pltpu.SemaphoreType.DMA] * CPB)
  return pl.kernel(
      functools.partial(_edge_agg_body, with_cnt, D),
      out_type=out_type, mesh=mesh, scratch_types=scratch,
      compiler_params=pltpu.CompilerParams(use_tc_tiling_on_sc=False),
      name=f"edge_agg_d{D}")


def _tc_a_body(x_ref, wl_ref, wr_ref, bl_ref, y_ref, r_ref):
  xb = x_ref[...]
  y_ref[...] = jnp.dot(xb, wl_ref[...], preferred_element_type=jnp.float32)
  r_ref[...] = (jnp.dot(xb, wr_ref[...], preferred_element_type=jnp.float32)
                + bl_ref[...])


def _tc_mid_body(p_ref, c_ref, r_ref, wl_ref, wr_ref, bl_ref,
                 y_ref, r2_ref):
  agg = p_ref[0] + p_ref[1]
  cnt = (c_ref[0] + c_ref[1])[:, 0:1]
  mean = agg / jnp.maximum(cnt, 1.0)
  h = jax.nn.relu(mean + r_ref[...])
  y_ref[...] = jnp.dot(h, wl_ref[...], preferred_element_type=jnp.float32)
  r2_ref[...] = (jnp.dot(h, wr_ref[...], preferred_element_type=jnp.float32)
                 + bl_ref[...])


def _tc_out_body(p_ref, c_ref, r_ref, wfc_ref, bfc_ref, o_ref):
  agg = p_ref[0] + p_ref[1]
  cnt = (c_ref[0] + c_ref[1])[:, 0:1]
  mean = agg / jnp.maximum(cnt, 1.0)
  h = jax.nn.relu(mean + r_ref[...])
  o_ref[...] = (jnp.dot(h, wfc_ref[...], preferred_element_type=jnp.float32)
                + bfc_ref[...])


def _row_spec(d):
  return pl.BlockSpec((BR, d), lambda i: (i, 0))


def _part_spec(d):
  return pl.BlockSpec((NC, BR, d), lambda i: (0, i, 0))


def _full_spec(a, b):
  return pl.BlockSpec((a, b), lambda i: (0, 0))


def kernel(x, edge_index, Wl1, bl1, Wr1, Wl2, bl2, Wr2, Wfc, bfc):
  ei = edge_index.astype(jnp.int32)
  pad = EPAD - E

  def worker_layout(flat):
    e0 = NS * NB0 * CPB * CH
    a0 = flat[:e0].reshape(NS, NB0 * CPB, CH)
    a1 = flat[e0:].reshape(NS, NB1 * CPB, CH)
    fill = jnp.zeros((NS, (NBMAX - NB1) * CPB, CH), jnp.int32)
    a0 = jnp.concatenate(
        [a0, jnp.zeros((NS, (NBMAX - NB0) * CPB, CH), jnp.int32)], axis=1)
    a1 = jnp.concatenate([a1, fill], axis=1)
    return jnp.concatenate([a0, a1], axis=0)

  src = worker_layout(jnp.concatenate([ei[0], jnp.zeros((pad,), jnp.int32)]))
  pad_dst = N + jnp.arange(pad, dtype=jnp.int32) % (N_SH - N)
  dst = worker_layout(jnp.concatenate([ei[1], pad_dst]))

  grid = N // BR

  y1, r1 = pl.pallas_call(
      _tc_a_body,
      grid=(grid,),
      in_specs=[_row_spec(128), _full_spec(128, 32), _full_spec(128, 32),
                _full_spec(1, 32)],
      out_specs=[_row_spec(32), _row_spec(32)],
      out_shape=[jax.ShapeDtypeStruct((N, 32), jnp.float32)] * 2,
  )(x, Wl1.T, Wr1.T, bl1.reshape(1, 32))

  p1, cnt = _make_edge_agg(32, True)(y1, src, dst)

  y2, r2 = pl.pallas_call(
      _tc_mid_body,
      grid=(grid,),
      in_specs=[_part_spec(32), _part_spec(L), _row_spec(32),
                _full_spec(32, 16), _full_spec(32, 16), _full_spec(1, 16)],
      out_specs=[_row_spec(16), _row_spec(16)],
      out_shape=[jax.ShapeDtypeStruct((N, 16), jnp.float32)] * 2,
  )(p1, cnt, r1, Wl2.T, Wr2.T, bl2.reshape(1, 16))

  (p2,) = _make_edge_agg(16, False)(y2, src, dst)

  out = pl.pallas_call(
      _tc_out_body,
      grid=(grid,),
      in_specs=[_part_spec(16), _part_spec(L), _row_spec(16),
                _full_spec(16, 2), _full_spec(1, 2)],
      out_specs=_row_spec(2),
      out_shape=jax.ShapeDtypeStruct((N, 2), jnp.float32),
  )(p2, cnt, r2, Wfc.T, bfc.reshape(1, 2))
  return out
